# Initial kernel scaffold; baseline (speedup 1.0000x reference)
import functools

import jax
import jax.numpy as jnp
from jax import lax
from jax.experimental import pallas as pl
from jax.experimental.pallas import tpu as pltpu
from jax.experimental.pallas import tpu_sc as plsc

P = 128
B = 128
H = 256
PB = P * B
ALPHA = 0.5
UNIF_CONST = (1.0 - ALPHA) / P

SCHUNK = 16
NSTEP = P // SCHUNK

_TINY = jnp.float32(1.1754943508222875e-38)
_ONE_MINUS_TINY = jnp.float32(1.0)


def _threefry2x32(x0, x1):
    k0 = jnp.uint32(0)
    k1 = jnp.uint32(42)
    ks = [k0, k1, k0 ^ k1 ^ jnp.uint32(0x1BD11BDA)]
    rot_groups = ((13, 15, 26, 6), (17, 29, 16, 24))

    x0 = x0 + ks[0]
    x1 = x1 + ks[1]
    for i in range(5):
        for r in rot_groups[i % 2]:
            x0 = x0 + x1
            x1 = (x1 << jnp.uint32(r)) | (x1 >> jnp.uint32(32 - r))
            x1 = x1 ^ x0
        x0 = x0 + ks[(i + 1) % 3]
        x1 = x1 + ks[(i + 2) % 3] + jnp.uint32(i + 1)
    return x0, x1


def _sample_body(prob_pb_ref, prob_bp_ref, idx_ref, pre_ref):
    t = pl.program_id(0)
    prob_pb = prob_pb_ref[...]
    prob_bp = prob_bp_ref[...]

    l_bp = jnp.log(ALPHA * jnp.exp(prob_bp) + UNIF_CONST)

    shape = (SCHUNK, B, P)
    s_i = lax.broadcasted_iota(jnp.uint32, shape, 0)
    b_i = lax.broadcasted_iota(jnp.uint32, shape, 1)
    j_i = lax.broadcasted_iota(jnp.uint32, shape, 2)
    base = (t * (SCHUNK * B * P)).astype(jnp.uint32)
    cnt = base + s_i * jnp.uint32(B * P) + b_i * jnp.uint32(P) + j_i
    o0, o1 = _threefry2x32(jnp.zeros(shape, jnp.uint32), cnt)
    bits = o0 ^ o1

    float_bits = (bits >> jnp.uint32(9)) | jnp.uint32(0x3F800000)
    floats = lax.bitcast_convert_type(float_bits, jnp.float32) - jnp.float32(1.0)
    u = jnp.maximum(_TINY, floats * _ONE_MINUS_TINY + _TINY)
    g = -jnp.log(-jnp.log(u))

    vals = g + l_bp[None, :, :]
    m = jnp.max(vals, axis=2, keepdims=True)
    jj = lax.broadcasted_iota(jnp.int32, shape, 2)
    cand = jnp.where(vals == m, jj, jnp.int32(P))
    idx = jnp.min(cand, axis=2)

    b2 = lax.broadcasted_iota(jnp.int32, (SCHUNK, B), 1)
    idx_ref[...] = b2 + idx * B

    l_pb = jnp.log(ALPHA * jnp.exp(prob_pb) + UNIF_CONST)
    d = prob_pb - l_pb
    jj2 = lax.broadcasted_iota(jnp.int32, (P, SCHUNK, B), 0)
    pre3 = jnp.where(idx[None, :, :] == jj2, d[:, None, :], jnp.float32(0.0))
    pre_ref[...] = jnp.sum(pre3, axis=0)


def _norm_body(pre_ref, out_ref):
    pre = pre_ref[...]
    m = jnp.max(pre, axis=0, keepdims=True)
    lse = m + jnp.log(jnp.sum(jnp.exp(pre - m), axis=0, keepdims=True))
    out_ref[...] = pre - lse


_sample_call = pl.pallas_call(
    _sample_body,
    grid=(NSTEP,),
    in_specs=[
        pl.BlockSpec((P, B), lambda t: (0, 0)),
        pl.BlockSpec((B, P), lambda t: (0, 0)),
    ],
    out_specs=[
        pl.BlockSpec((SCHUNK, B), lambda t: (t, 0)),
        pl.BlockSpec((SCHUNK, B), lambda t: (t, 0)),
    ],
    out_shape=[
        jax.ShapeDtypeStruct((P, B), jnp.int32),
        jax.ShapeDtypeStruct((P, B), jnp.float32),
    ],
)

_norm_call = pl.pallas_call(
    _norm_body,
    out_shape=jax.ShapeDtypeStruct((P, B), jnp.float32),
)

NC = 2
NS = 16
NW = NC * NS
ROWS_W = PB // NW
CH = 128
NCHUNK = ROWS_W // CH
IDX_ROWS_W = ROWS_W // B

_sc_mesh = plsc.VectorSubcoreMesh(
    core_axis_name="c", subcore_axis_name="s", num_cores=NC, num_subcores=NS
)


@functools.partial(
    pl.kernel,
    mesh=_sc_mesh,
    out_type=jax.ShapeDtypeStruct((PB, H), jnp.float32),
    scratch_types=[
        pltpu.VMEM((NCHUNK, CH), jnp.int32),
        pltpu.VMEM((2, CH, H), jnp.float32),
        pltpu.SemaphoreType.DMA,
        pltpu.SemaphoreType.DMA,
    ],
)
def _gather_call(idx_hbm, parts_hbm, out_hbm, idx_v, buf_v, sem0, sem1):
    wid = lax.axis_index("s") * NC + lax.axis_index("c")
    pltpu.sync_copy(idx_hbm.at[pl.ds(wid * IDX_ROWS_W, IDX_ROWS_W)], idx_v)
    sems = (sem0, sem1)
    copies = [None, None]
    copies[0] = pltpu.async_copy(parts_hbm.at[idx_v.at[0]], buf_v.at[0], sem0)
    for c in range(NCHUNK):
        nxt = c + 1
        if nxt < NCHUNK:
            copies[nxt % 2] = pltpu.async_copy(
                parts_hbm.at[idx_v.at[nxt]], buf_v.at[nxt % 2], sems[nxt % 2]
            )
        copies[c % 2].wait()
        pltpu.sync_copy(
            buf_v.at[c % 2], out_hbm.at[pl.ds(wid * ROWS_W + c * CH, CH)]
        )


def kernel(particles, prob):
    prob_pb = prob.reshape(P, B)
    prob_bp = prob_pb.T
    flat_idx, pre = _sample_call(prob_pb, prob_bp)
    prob_new = _norm_call(pre)
    particles_new = _gather_call(flat_idx, particles)
    return particles_new, prob_new.reshape(P, B, 1)


# speedup vs baseline: 1.2324x; 1.2324x over previous
import functools

import jax
import jax.numpy as jnp
from jax import lax
from jax.experimental import pallas as pl
from jax.experimental.pallas import tpu as pltpu
from jax.experimental.pallas import tpu_sc as plsc

P = 128
B = 128
H = 256
PB = P * B
ALPHA = 0.5
UNIF_CONST = (1.0 - ALPHA) / P

import numpy as np

_TINY = np.float32(1.1754943508222875e-38)
_ONE_MINUS_TINY = np.float32(1.0)


def _threefry2x32(x0, x1):
    k0 = jnp.uint32(0)
    k1 = jnp.uint32(42)
    ks = [k0, k1, k0 ^ k1 ^ jnp.uint32(0x1BD11BDA)]
    rot_groups = ((13, 15, 26, 6), (17, 29, 16, 24))

    x0 = x0 + ks[0]
    x1 = x1 + ks[1]
    for i in range(5):
        for r in rot_groups[i % 2]:
            x0 = x0 + x1
            x1 = (x1 << jnp.uint32(r)) | (x1 >> jnp.uint32(32 - r))
            x1 = x1 ^ x0
        x0 = x0 + ks[(i + 1) % 3]
        x1 = x1 + ks[(i + 2) % 3] + jnp.uint32(i + 1)
    return x0, x1


def _sample_body(prob_pb_ref, idx_ref, pre_ref):
    prob_pb = prob_pb_ref[...]

    l_pb = jnp.log(ALPHA * jnp.exp(prob_pb) + UNIF_CONST)
    d = prob_pb - l_pb

    j_i = lax.broadcasted_iota(jnp.uint32, (P, B), 0)
    b_i = lax.broadcasted_iota(jnp.uint32, (P, B), 1)
    cnt0 = b_i * jnp.uint32(P) + j_i
    jcol = lax.broadcasted_iota(jnp.int32, (P, B), 0)
    bline = lax.broadcasted_iota(jnp.int32, (1, B), 1)

    def step(s, _):
        cnt = cnt0 + (s * (B * P)).astype(jnp.uint32)
        o0, o1 = _threefry2x32(jnp.zeros((P, B), jnp.uint32), cnt)
        bits = o0 ^ o1

        float_bits = (bits >> jnp.uint32(9)) | jnp.uint32(0x3F800000)
        floats = lax.bitcast_convert_type(float_bits, jnp.float32)
        floats = floats - jnp.float32(1.0)
        u = jnp.maximum(_TINY, floats * _ONE_MINUS_TINY + _TINY)
        g = -jnp.log(-jnp.log(u))

        vals = g + l_pb
        m = jnp.max(vals, axis=0, keepdims=True)
        cand = jnp.where(vals == m, jcol, jnp.int32(P))
        idx_row = jnp.min(cand, axis=0, keepdims=True)

        idx_ref[pl.ds(s, 1), :] = bline + idx_row * B

        mask = idx_row == jcol
        pre_ref[pl.ds(s, 1), :] = jnp.sum(
            jnp.where(mask, d, jnp.float32(0.0)), axis=0, keepdims=True
        )
        return 0

    lax.fori_loop(0, P, step, 0)


def _norm_body(pre_ref, out_ref):
    pre = pre_ref[...]
    m = jnp.max(pre, axis=0, keepdims=True)
    lse = m + jnp.log(jnp.sum(jnp.exp(pre - m), axis=0, keepdims=True))
    out_ref[...] = pre - lse


_sample_call = pl.pallas_call(
    _sample_body,
    out_shape=[
        jax.ShapeDtypeStruct((P, B), jnp.int32),
        jax.ShapeDtypeStruct((P, B), jnp.float32),
    ],
)

_norm_call = pl.pallas_call(
    _norm_body,
    out_shape=jax.ShapeDtypeStruct((P, B), jnp.float32),
)

NC = 2
NS = 16
NW = NC * NS
ROWS_W = PB // NW
CH = 128
NCHUNK = ROWS_W // CH
IDX_ROWS_W = ROWS_W // B

@functools.lru_cache(maxsize=None)
def _make_gather_call():
    mesh = plsc.VectorSubcoreMesh(
        core_axis_name="c", subcore_axis_name="s", num_cores=NC, num_subcores=NS
    )

    @functools.partial(
        pl.kernel,
        mesh=mesh,
        out_type=jax.ShapeDtypeStruct((PB, H), jnp.float32),
        scratch_types=[
            pltpu.VMEM((NCHUNK, CH), jnp.int32),
            pltpu.VMEM((2, CH, H), jnp.float32),
            pltpu.SemaphoreType.DMA,
            pltpu.SemaphoreType.DMA,
        ],
    )
    def gather_call(idx_hbm, parts_hbm, out_hbm, idx_v, buf_v, sem0, sem1):
        wid = lax.axis_index("s") * NC + lax.axis_index("c")
        pltpu.sync_copy(idx_hbm.at[pl.ds(wid * IDX_ROWS_W, IDX_ROWS_W)], idx_v)
        sems = (sem0, sem1)
        copies = [None, None]
        copies[0] = pltpu.async_copy(parts_hbm.at[idx_v.at[0]], buf_v.at[0], sem0)
        for c in range(NCHUNK):
            nxt = c + 1
            if nxt < NCHUNK:
                copies[nxt % 2] = pltpu.async_copy(
                    parts_hbm.at[idx_v.at[nxt]], buf_v.at[nxt % 2], sems[nxt % 2]
                )
            copies[c % 2].wait()
            pltpu.sync_copy(
                buf_v.at[c % 2], out_hbm.at[pl.ds(wid * ROWS_W + c * CH, CH)]
            )

    return gather_call


def kernel(particles, prob):
    prob_pb = prob.reshape(P, B)
    flat_idx, pre = _sample_call(prob_pb)
    prob_new = _norm_call(pre)
    particles_new = _make_gather_call()(flat_idx, particles)
    return particles_new, prob_new.reshape(P, B, 1)


# speedup vs baseline: 1.2752x; 1.0348x over previous
import functools

import jax
import jax.numpy as jnp
from jax import lax
from jax.experimental import pallas as pl
from jax.experimental.pallas import tpu as pltpu
from jax.experimental.pallas import tpu_sc as plsc

P = 128
B = 128
H = 256
PB = P * B
ALPHA = 0.5
UNIF_CONST = (1.0 - ALPHA) / P

import numpy as np

_TINY = np.float32(1.1754943508222875e-38)
_ONE_MINUS_TINY = np.float32(1.0)


def _threefry2x32_zero(cnt):
    k0 = jnp.uint32(0)
    k1 = jnp.uint32(42)
    ks = [k0, k1, k0 ^ k1 ^ jnp.uint32(0x1BD11BDA)]
    rot_groups = ((13, 15, 26, 6), (17, 29, 16, 24))

    x1 = cnt + ks[1]
    x0 = x1
    first = True
    for i in range(5):
        for r in rot_groups[i % 2]:
            if first:
                first = False
            else:
                x0 = x0 + x1
            x1 = (x1 << jnp.uint32(r)) | (x1 >> jnp.uint32(32 - r))
            x1 = x1 ^ x0
        x0 = x0 + ks[(i + 1) % 3]
        x1 = x1 + ks[(i + 2) % 3] + jnp.uint32(i + 1)
    return x0, x1


def _sample_body(prob_pb_ref, idx_ref, pre_ref):
    prob_pb = prob_pb_ref[...]

    l_pb = jnp.log(ALPHA * jnp.exp(prob_pb) + UNIF_CONST)
    d = prob_pb - l_pb

    j_i = lax.broadcasted_iota(jnp.uint32, (P, B), 0)
    b_i = lax.broadcasted_iota(jnp.uint32, (P, B), 1)
    cnt0 = b_i * jnp.uint32(P) + j_i
    jcol = lax.broadcasted_iota(jnp.int32, (P, B), 0)
    bline = lax.broadcasted_iota(jnp.int32, (1, B), 1)

    def one_row(s):
        cnt = cnt0 + (s * (B * P)).astype(jnp.uint32)
        o0, o1 = _threefry2x32_zero(cnt)
        bits = o0 ^ o1

        float_bits = (bits >> jnp.uint32(9)) | jnp.uint32(0x3F800000)
        floats = lax.bitcast_convert_type(float_bits, jnp.float32)
        floats = floats - jnp.float32(1.0)
        u = jnp.maximum(_TINY, floats * _ONE_MINUS_TINY + _TINY)
        g = -jnp.log(-jnp.log(u))

        vals = g + l_pb
        m = jnp.max(vals, axis=0, keepdims=True)
        cand = jnp.where(vals == m, jcol, jnp.int32(P))
        idx_row = jnp.min(cand, axis=0, keepdims=True)

        idx_ref[pl.ds(s, 1), :] = bline + idx_row * B

        mask = idx_row == jcol
        pre_ref[pl.ds(s, 1), :] = jnp.sum(
            jnp.where(mask, d, jnp.float32(0.0)), axis=0, keepdims=True
        )

    def step(k, _):
        one_row(2 * k)
        one_row(2 * k + 1)
        return 0

    lax.fori_loop(0, P // 2, step, 0)


def _norm_body(pre_ref, out_ref):
    pre = pre_ref[...]
    m = jnp.max(pre, axis=0, keepdims=True)
    lse = m + jnp.log(jnp.sum(jnp.exp(pre - m), axis=0, keepdims=True))
    out_ref[...] = pre - lse


_sample_call = pl.pallas_call(
    _sample_body,
    out_shape=[
        jax.ShapeDtypeStruct((P, B), jnp.int32),
        jax.ShapeDtypeStruct((P, B), jnp.float32),
    ],
)

_norm_call = pl.pallas_call(
    _norm_body,
    out_shape=jax.ShapeDtypeStruct((P, B), jnp.float32),
)

NC = 2
NS = 16
NW = NC * NS
ROWS_W = PB // NW
CH = 128
NCHUNK = ROWS_W // CH
IDX_ROWS_W = ROWS_W // B

@functools.lru_cache(maxsize=None)
def _make_gather_call():
    mesh = plsc.VectorSubcoreMesh(
        core_axis_name="c", subcore_axis_name="s", num_cores=NC, num_subcores=NS
    )

    @functools.partial(
        pl.kernel,
        mesh=mesh,
        out_type=jax.ShapeDtypeStruct((PB, H), jnp.float32),
        scratch_types=[
            pltpu.VMEM((NCHUNK, CH), jnp.int32),
            pltpu.VMEM((2, CH, H), jnp.float32),
            pltpu.SemaphoreType.DMA,
            pltpu.SemaphoreType.DMA,
        ],
    )
    def gather_call(idx_hbm, parts_hbm, out_hbm, idx_v, buf_v, sem0, sem1):
        wid = lax.axis_index("s") * NC + lax.axis_index("c")
        pltpu.sync_copy(idx_hbm.at[pl.ds(wid * IDX_ROWS_W, IDX_ROWS_W)], idx_v)
        sems = (sem0, sem1)
        copies = [None, None]
        copies[0] = pltpu.async_copy(parts_hbm.at[idx_v.at[0]], buf_v.at[0], sem0)
        for c in range(NCHUNK):
            nxt = c + 1
            if nxt < NCHUNK:
                copies[nxt % 2] = pltpu.async_copy(
                    parts_hbm.at[idx_v.at[nxt]], buf_v.at[nxt % 2], sems[nxt % 2]
                )
            copies[c % 2].wait()
            pltpu.sync_copy(
                buf_v.at[c % 2], out_hbm.at[pl.ds(wid * ROWS_W + c * CH, CH)]
            )

    return gather_call


def kernel(particles, prob):
    prob_pb = prob.reshape(P, B)
    flat_idx, pre = _sample_call(prob_pb)
    prob_new = _norm_call(pre)
    particles_new = _make_gather_call()(flat_idx, particles)
    return particles_new, prob_new.reshape(P, B, 1)
